# baseline (speedup 1.0000x reference)
import functools

import jax
import jax.numpy as jnp
from jax import lax
from jax.experimental import pallas as pl
from jax.experimental.pallas import tpu as pltpu
from jax.experimental.pallas import tpu_sc as plsc

N = 50000
K = 6
D = 128
L = 5
IN_DIM = 3

BN = 1024
NP = 49 * BN
GRID = NP // BN

NC = 2
NS = 16
NW = NC * NS
NODES_PER_W = NP // NW
CHUNK = 16
NUM_CHUNKS = NODES_PER_W // CHUNK
ROWS_PER_CHUNK = CHUNK * K


RING = 4
IDX_CHUNKS = 104


def _gather_sum_sc(table, idx3, zrows, didx):
    mesh = plsc.VectorSubcoreMesh(core_axis_name="c", subcore_axis_name="s")

    @functools.partial(
        pl.kernel,
        mesh=mesh,
        out_type=jax.ShapeDtypeStruct((NP, D), jnp.float32),
        scratch_types=[
            pltpu.VMEM((IDX_CHUNKS, ROWS_PER_CHUNK), jnp.int32),
            pltpu.VMEM((ROWS_PER_CHUNK, D), jnp.float32),
            pltpu.VMEM((ROWS_PER_CHUNK, D), jnp.float32),
            pltpu.VMEM((ROWS_PER_CHUNK, D), jnp.float32),
            pltpu.VMEM((ROWS_PER_CHUNK, D), jnp.float32),
            pltpu.VMEM((CHUNK, D), jnp.float32),
            pltpu.VMEM((CHUNK, D), jnp.float32),
            pltpu.SemaphoreType.DMA,
            pltpu.SemaphoreType.DMA,
            pltpu.SemaphoreType.DMA,
            pltpu.SemaphoreType.DMA,
        ],
    )
    def body(table_hbm, idx_hbm, zrows_hbm, didx_hbm, out_hbm, idx_v,
             rows0, rows1, rows2, rows3, acc0, acc1,
             sem0, sem1, sem2, sem3):
        wid = lax.axis_index("s") * NC + lax.axis_index("c")
        base = wid * NODES_PER_W
        rows = (rows0, rows1, rows2, rows3)
        accs = (acc0, acc1)
        sems = (sem0, sem1, sem2, sem3)

        pltpu.sync_copy(idx_hbm.at[wid], idx_v)
        for p in range(RING):
            pltpu.make_async_copy(
                table_hbm.at[idx_v.at[p]], rows[p], sems[p]).start()

        def reduce_chunk(rows_v, acc_v, nb):
            for nn in range(CHUNK):
                r0 = nn * K
                for c in range(0, D, 16):
                    acc = rows_v[r0, pl.ds(c, 16)]
                    for kk in range(1, K):
                        acc = acc + rows_v[r0 + kk, pl.ds(c, 16)]
                    acc_v[nn, pl.ds(c, 16)] = acc
            pltpu.sync_copy(acc_v, out_hbm.at[pl.ds(nb, CHUNK)])

        def step(j, carry):
            for p in range(RING):
                ci = RING * j + p
                pltpu.make_async_copy(
                    table_hbm.at[idx_v.at[ci]], rows[p], sems[p]).wait()

                @pl.when(ci < NUM_CHUNKS)
                def _():
                    reduce_chunk(rows[p], accs[p % 2], base + ci * CHUNK)

                pltpu.make_async_copy(
                    table_hbm.at[idx_v.at[ci + RING]], rows[p],
                    sems[p]).start()
            return carry

        lax.fori_loop(0, 25, step, 0)
        for p in range(RING):
            pltpu.make_async_copy(
                table_hbm.at[idx_v.at[100 + p]], rows[p], sems[p]).wait()

    return body(table, idx3, zrows, didx)


def _ln(h, g, b):
    m = jnp.mean(h, axis=-1, keepdims=True)
    c = h - m
    v = jnp.mean(c * c, axis=-1, keepdims=True)
    return c * lax.rsqrt(v + 1e-5) * g + b


def _row_spec():
    return pl.BlockSpec((BN, D), lambda i: (i, 0))


def _full_spec(shape):
    nd = len(shape)
    return pl.BlockSpec(shape, lambda i: (0,) * nd)


def _emb_body(x_ref, We_ref, be_ref, g_ref, b_ref, h_ref, ln_ref):
    h = jnp.dot(x_ref[...], We_ref[...], preferred_element_type=jnp.float32)
    h = h + be_ref[...]
    h_ref[...] = h
    ln_ref[...] = _ln(h, g_ref[...], b_ref[...])


def _embed(xp, Wep, be, g0, b0):
    return pl.pallas_call(
        _emb_body,
        grid=(GRID,),
        in_specs=[
            pl.BlockSpec((BN, 8), lambda i: (i, 0)),
            _full_spec((8, D)),
            _full_spec((1, D)),
            _full_spec((1, D)),
            _full_spec((1, D)),
        ],
        out_specs=[_row_spec(), _row_spec()],
        out_shape=[
            jax.ShapeDtypeStruct((NP, D), jnp.float32),
            jax.ShapeDtypeStruct((NP, D), jnp.float32),
        ],
    )(xp, Wep, be, g0, b0)


def _mid_body(h_ref, s_ref, Wmp_ref, bmp_ref, g_ref, b_ref, Wqkv_ref,
              h2_ref, q_ref, kv_ref, ksum_ref):
    i = pl.program_id(0)
    s = s_ref[...] * (1.0 / K)
    h2 = h_ref[...] + jnp.dot(s, Wmp_ref[...], preferred_element_type=jnp.float32)
    h2 = h2 + bmp_ref[...]
    h2_ref[...] = h2
    ln = _ln(h2, g_ref[...], b_ref[...])
    qkv = jnp.dot(ln, Wqkv_ref[...], preferred_element_type=jnp.float32)
    q = jax.nn.relu(qkv[:, :D]) + 1e-6
    k = jax.nn.relu(qkv[:, D:2 * D]) + 1e-6
    v = qkv[:, 2 * D:]
    rows = i * BN + lax.broadcasted_iota(jnp.int32, (BN, 1), 0)
    k = jnp.where(rows < N, k, 0.0)
    q_ref[...] = q.astype(jnp.bfloat16)
    kv_c = lax.dot_general(k, v, (((0,), (0,)), ((), ())),
                           preferred_element_type=jnp.float32)
    ksum_c = jnp.sum(k, axis=0, keepdims=True)

    @pl.when(i == 0)
    def _():
        kv_ref[...] = kv_c
        ksum_ref[...] = ksum_c

    @pl.when(i > 0)
    def _():
        kv_ref[...] += kv_c
        ksum_ref[...] += ksum_c


def _mid(h, s, Wmp_i, bmp_i, g, b, Wqkv_i):
    return pl.pallas_call(
        _mid_body,
        grid=(GRID,),
        in_specs=[
            _row_spec(), _row_spec(),
            _full_spec((D, D)), _full_spec((1, D)),
            _full_spec((1, D)), _full_spec((1, D)),
            _full_spec((D, 3 * D)),
        ],
        out_specs=[
            _row_spec(), _row_spec(),
            _full_spec((D, D)), _full_spec((1, D)),
        ],
        out_shape=[
            jax.ShapeDtypeStruct((NP, D), jnp.float32),
            jax.ShapeDtypeStruct((NP, D), jnp.bfloat16),
            jax.ShapeDtypeStruct((D, D), jnp.float32),
            jax.ShapeDtypeStruct((1, D), jnp.float32),
        ],
    )(h, s, Wmp_i, bmp_i, g, b, Wqkv_i)


def _attn_core(q, kv, ksum):
    q = q.astype(jnp.float32)
    z = 1.0 / (jnp.sum(q * ksum, axis=1, keepdims=True) + 1e-6)
    return jnp.dot(q, kv, preferred_element_type=jnp.float32) * z


def _post_body(h2_ref, q_ref, kv_ref, ksum_ref, Wout_ref, bout_ref, g_ref, b_ref,
               h3_ref, ln_ref):
    attn = _attn_core(q_ref[...], kv_ref[...], ksum_ref[...])
    h3 = h2_ref[...] + jnp.dot(attn, Wout_ref[...],
                               preferred_element_type=jnp.float32)
    h3 = h3 + bout_ref[...]
    h3_ref[...] = h3
    ln_ref[...] = _ln(h3, g_ref[...], b_ref[...])


def _post(h2, q, kv, ksum, Wout_i, bout_i, g_next, b_next):
    return pl.pallas_call(
        _post_body,
        grid=(GRID,),
        in_specs=[
            _row_spec(), _row_spec(),
            _full_spec((D, D)), _full_spec((1, D)),
            _full_spec((D, D)), _full_spec((1, D)),
            _full_spec((1, D)), _full_spec((1, D)),
        ],
        out_specs=[_row_spec(), _row_spec()],
        out_shape=[
            jax.ShapeDtypeStruct((NP, D), jnp.float32),
            jax.ShapeDtypeStruct((NP, D), jnp.float32),
        ],
    )(h2, q, kv, ksum, Wout_i, bout_i, g_next, b_next)


def _final_body(h2_ref, q_ref, kv_ref, ksum_ref, Wout_ref, bout_ref,
                Whead_ref, bhead_ref, out_ref):
    attn = _attn_core(q_ref[...], kv_ref[...], ksum_ref[...])
    h3 = h2_ref[...] + jnp.dot(attn, Wout_ref[...],
                               preferred_element_type=jnp.float32)
    h3 = h3 + bout_ref[...]
    out_ref[...] = jnp.dot(h3, Whead_ref[...],
                           preferred_element_type=jnp.float32) + bhead_ref[...]


def _final(h2, q, kv, ksum, Wout_i, bout_i, Whead_p, bhead_p):
    return pl.pallas_call(
        _final_body,
        grid=(GRID,),
        in_specs=[
            _row_spec(), _row_spec(),
            _full_spec((D, D)), _full_spec((1, D)),
            _full_spec((D, D)), _full_spec((1, D)),
            _full_spec((D, 8)), _full_spec((1, 8)),
        ],
        out_specs=[pl.BlockSpec((BN, 8), lambda i: (i, 0))],
        out_shape=[jax.ShapeDtypeStruct((NP, 8), jnp.float32)],
    )(h2, q, kv, ksum, Wout_i, bout_i, Whead_p, bhead_p)


def _gather_sum(table, idx, zrows, didx):
    return _gather_sum_sc(table, idx, zrows, didx)


def kernel(x, knn, W_emb, b_emb, ln_g, ln_b, Wmp, bmp, Wqkv, Wout, bout,
           Whead, bhead):
    x2 = x.reshape(N, IN_DIM)
    xp = jnp.pad(x2, ((0, NP - N), (0, 8 - IN_DIM)))
    Wep = jnp.pad(W_emb, ((0, 8 - IN_DIM), (0, 0)))
    idx = jnp.pad(knn.reshape(N * K), (0, (NP - N) * K))
    idx3 = jnp.pad(
        idx.reshape(NW, NUM_CHUNKS, ROWS_PER_CHUNK),
        ((0, 0), (0, IDX_CHUNKS - NUM_CHUNKS), (0, 0)))
    Whead_p = jnp.pad(Whead, ((0, 0), (0, 8 - Whead.shape[1])))
    bhead_p = jnp.pad(bhead, (0, 8 - bhead.shape[0])).reshape(1, 8)

    r = lambda a: a.reshape(1, D)
    zrows = jnp.zeros((CHUNK, D), jnp.float32)
    didx = (jnp.arange(ROWS_PER_CHUNK, dtype=jnp.int32) // K)
    h, ln1 = _embed(xp, Wep, b_emb.reshape(1, D), r(ln_g[0]), r(ln_b[0]))
    for i in range(L):
        s = _gather_sum(ln1, idx3, zrows, didx)
        h, q, kv, ksum = _mid(h, s, Wmp[i], r(bmp[i]),
                              r(ln_g[2 * i + 1]), r(ln_b[2 * i + 1]), Wqkv[i])
        if i + 1 < L:
            h, ln1 = _post(h, q, kv, ksum, Wout[i], r(bout[i]),
                           r(ln_g[2 * i + 2]), r(ln_b[2 * i + 2]))
        else:
            outp = _final(h, q, kv, ksum, Wout[i], r(bout[i]),
                          Whead_p, bhead_p)[0]
    return outp[:N, :3].reshape(1, N, 3)


# speedup vs baseline: 1.8948x; 1.8948x over previous
import functools

import jax
import jax.numpy as jnp
from jax import lax
from jax.experimental import pallas as pl
from jax.experimental.pallas import tpu as pltpu
from jax.experimental.pallas import tpu_sc as plsc

N = 50000
K = 6
D = 128
L = 5
IN_DIM = 3

BN = 1008
NP = 50 * BN
GRID = NP // BN

NC = 2
NS = 16
NW = NC * NS
NODES_PER_W = NP // NW
CHUNK = 21
NUM_CHUNKS = NODES_PER_W // CHUNK
ROWS_PER_CHUNK = CHUNK * K
IDX_CHUNKS = NUM_CHUNKS


def _gather_sum_sc(table, idx3, zrows, didx):
    mesh = plsc.VectorSubcoreMesh(core_axis_name="c", subcore_axis_name="s")

    @functools.partial(
        pl.kernel,
        mesh=mesh,
        out_type=jax.ShapeDtypeStruct((NP * D,), jnp.float32),
        scratch_types=[
            pltpu.VMEM((ROWS_PER_CHUNK,), jnp.int32),
            pltpu.VMEM((ROWS_PER_CHUNK, D), jnp.float32),
            pltpu.VMEM((CHUNK * D,), jnp.float32),
            pltpu.SemaphoreType.DMA,
        ],
    )
    def body(table_hbm, idx_hbm, zrows_hbm, didx_hbm, out_hbm, idx_v,
             rows_v, acc_v, sem):
        wid = lax.axis_index("s") * NC + lax.axis_index("c")
        base = wid * NODES_PER_W

        def step(ci, carry):
            nb = base + ci * CHUNK
            pltpu.sync_copy(idx_hbm.at[wid].at[ci], idx_v)
            pltpu.make_async_copy(table_hbm.at[idx_v], rows_v, sem).start()
            pltpu.make_async_copy(table_hbm.at[idx_v], rows_v, sem).wait()
            for nn in range(CHUNK):
                r0 = nn * K
                for c in range(0, D, 16):
                    acc = rows_v[r0, pl.ds(c, 16)]
                    for kk in range(1, K):
                        acc = acc + rows_v[r0 + kk, pl.ds(c, 16)]
                    acc_v[pl.ds(nn * D + c, 16)] = acc
            pltpu.sync_copy(acc_v, out_hbm.at[pl.ds(nb * D, CHUNK * D)])
            return carry

        lax.fori_loop(0, NUM_CHUNKS, step, 0)

    return body(table, idx3, zrows, didx)


def _ln(h, g, b):
    m = jnp.mean(h, axis=-1, keepdims=True)
    c = h - m
    v = jnp.mean(c * c, axis=-1, keepdims=True)
    return c * lax.rsqrt(v + 1e-5) * g + b


def _row_spec():
    return pl.BlockSpec((BN, D), lambda i: (i, 0))


def _full_spec(shape):
    nd = len(shape)
    return pl.BlockSpec(shape, lambda i: (0,) * nd)


def _emb_body(x_ref, We_ref, be_ref, g_ref, b_ref, h_ref, ln_ref):
    h = jnp.dot(x_ref[...], We_ref[...], preferred_element_type=jnp.float32)
    h = h + be_ref[...]
    h_ref[...] = h
    ln_ref[...] = _ln(h, g_ref[...], b_ref[...])


def _embed(xp, Wep, be, g0, b0):
    return pl.pallas_call(
        _emb_body,
        grid=(GRID,),
        in_specs=[
            pl.BlockSpec((BN, 8), lambda i: (i, 0)),
            _full_spec((8, D)),
            _full_spec((1, D)),
            _full_spec((1, D)),
            _full_spec((1, D)),
        ],
        out_specs=[_row_spec(), _row_spec()],
        out_shape=[
            jax.ShapeDtypeStruct((NP, D), jnp.float32),
            jax.ShapeDtypeStruct((NP, D), jnp.float32),
        ],
    )(xp, Wep, be, g0, b0)


def _mid_body(h_ref, s_ref, Wmp_ref, bmp_ref, g_ref, b_ref, Wqkv_ref,
              h2_ref, q_ref, kv_ref, ksum_ref):
    i = pl.program_id(0)
    s = s_ref[...] * (1.0 / K)
    h2 = h_ref[...] + jnp.dot(s, Wmp_ref[...], preferred_element_type=jnp.float32)
    h2 = h2 + bmp_ref[...]
    h2_ref[...] = h2
    ln = _ln(h2, g_ref[...], b_ref[...])
    qkv = jnp.dot(ln, Wqkv_ref[...], preferred_element_type=jnp.float32)
    q = jax.nn.relu(qkv[:, :D]) + 1e-6
    k = jax.nn.relu(qkv[:, D:2 * D]) + 1e-6
    v = qkv[:, 2 * D:]
    rows = i * BN + lax.broadcasted_iota(jnp.int32, (BN, 1), 0)
    k = jnp.where(rows < N, k, 0.0)
    q_ref[...] = q.astype(jnp.bfloat16)
    kv_c = lax.dot_general(k, v, (((0,), (0,)), ((), ())),
                           preferred_element_type=jnp.float32)
    ksum_c = jnp.sum(k, axis=0, keepdims=True)

    @pl.when(i == 0)
    def _():
        kv_ref[...] = kv_c
        ksum_ref[...] = ksum_c

    @pl.when(i > 0)
    def _():
        kv_ref[...] += kv_c
        ksum_ref[...] += ksum_c


def _mid(h, s, Wmp_i, bmp_i, g, b, Wqkv_i):
    return pl.pallas_call(
        _mid_body,
        grid=(GRID,),
        in_specs=[
            _row_spec(), _row_spec(),
            _full_spec((D, D)), _full_spec((1, D)),
            _full_spec((1, D)), _full_spec((1, D)),
            _full_spec((D, 3 * D)),
        ],
        out_specs=[
            _row_spec(), _row_spec(),
            _full_spec((D, D)), _full_spec((1, D)),
        ],
        out_shape=[
            jax.ShapeDtypeStruct((NP, D), jnp.float32),
            jax.ShapeDtypeStruct((NP, D), jnp.bfloat16),
            jax.ShapeDtypeStruct((D, D), jnp.float32),
            jax.ShapeDtypeStruct((1, D), jnp.float32),
        ],
    )(h, s, Wmp_i, bmp_i, g, b, Wqkv_i)


def _attn_core(q, kv, ksum):
    q = q.astype(jnp.float32)
    z = 1.0 / (jnp.sum(q * ksum, axis=1, keepdims=True) + 1e-6)
    return jnp.dot(q, kv, preferred_element_type=jnp.float32) * z


def _post_body(h2_ref, q_ref, kv_ref, ksum_ref, Wout_ref, bout_ref, g_ref, b_ref,
               h3_ref, ln_ref):
    attn = _attn_core(q_ref[...], kv_ref[...], ksum_ref[...])
    h3 = h2_ref[...] + jnp.dot(attn, Wout_ref[...],
                               preferred_element_type=jnp.float32)
    h3 = h3 + bout_ref[...]
    h3_ref[...] = h3
    ln_ref[...] = _ln(h3, g_ref[...], b_ref[...])


def _post(h2, q, kv, ksum, Wout_i, bout_i, g_next, b_next):
    return pl.pallas_call(
        _post_body,
        grid=(GRID,),
        in_specs=[
            _row_spec(), _row_spec(),
            _full_spec((D, D)), _full_spec((1, D)),
            _full_spec((D, D)), _full_spec((1, D)),
            _full_spec((1, D)), _full_spec((1, D)),
        ],
        out_specs=[_row_spec(), _row_spec()],
        out_shape=[
            jax.ShapeDtypeStruct((NP, D), jnp.float32),
            jax.ShapeDtypeStruct((NP, D), jnp.float32),
        ],
    )(h2, q, kv, ksum, Wout_i, bout_i, g_next, b_next)


def _final_body(h2_ref, q_ref, kv_ref, ksum_ref, Wout_ref, bout_ref,
                Whead_ref, bhead_ref, out_ref):
    attn = _attn_core(q_ref[...], kv_ref[...], ksum_ref[...])
    h3 = h2_ref[...] + jnp.dot(attn, Wout_ref[...],
                               preferred_element_type=jnp.float32)
    h3 = h3 + bout_ref[...]
    out_ref[...] = jnp.dot(h3, Whead_ref[...],
                           preferred_element_type=jnp.float32) + bhead_ref[...]


def _final(h2, q, kv, ksum, Wout_i, bout_i, Whead_p, bhead_p):
    return pl.pallas_call(
        _final_body,
        grid=(GRID,),
        in_specs=[
            _row_spec(), _row_spec(),
            _full_spec((D, D)), _full_spec((1, D)),
            _full_spec((D, D)), _full_spec((1, D)),
            _full_spec((D, 8)), _full_spec((1, 8)),
        ],
        out_specs=[pl.BlockSpec((BN, 8), lambda i: (i, 0))],
        out_shape=[jax.ShapeDtypeStruct((NP, 8), jnp.float32)],
    )(h2, q, kv, ksum, Wout_i, bout_i, Whead_p, bhead_p)


def _gather_sum(table, idx, zrows, didx):
    return _gather_sum_sc(table, idx, zrows, didx)


def kernel(x, knn, W_emb, b_emb, ln_g, ln_b, Wmp, bmp, Wqkv, Wout, bout,
           Whead, bhead):
    x2 = x.reshape(N, IN_DIM)
    xp = jnp.pad(x2, ((0, NP - N), (0, 8 - IN_DIM)))
    Wep = jnp.pad(W_emb, ((0, 8 - IN_DIM), (0, 0)))
    idx = jnp.pad(knn.reshape(N * K), (0, (NP - N) * K))
    idx3 = jnp.pad(
        idx.reshape(NW, NUM_CHUNKS, ROWS_PER_CHUNK),
        ((0, 0), (0, IDX_CHUNKS - NUM_CHUNKS), (0, 0)))
    Whead_p = jnp.pad(Whead, ((0, 0), (0, 8 - Whead.shape[1])))
    bhead_p = jnp.pad(bhead, (0, 8 - bhead.shape[0])).reshape(1, 8)

    r = lambda a: a.reshape(1, D)
    zrows = jnp.zeros((CHUNK, D), jnp.float32)
    didx = (jnp.arange(ROWS_PER_CHUNK, dtype=jnp.int32) // K)
    h, ln1 = _embed(xp, Wep, b_emb.reshape(1, D), r(ln_g[0]), r(ln_b[0]))
    for i in range(L):
        s = _gather_sum(ln1, idx3, zrows, didx).reshape(NP, D)
        h, q, kv, ksum = _mid(h, s, Wmp[i], r(bmp[i]),
                              r(ln_g[2 * i + 1]), r(ln_b[2 * i + 1]), Wqkv[i])
        if i + 1 < L:
            h, ln1 = _post(h, q, kv, ksum, Wout[i], r(bout[i]),
                           r(ln_g[2 * i + 2]), r(ln_b[2 * i + 2]))
        else:
            outp = _final(h, q, kv, ksum, Wout[i], r(bout[i]),
                          Whead_p, bhead_p)[0]
    return outp[:N, :3].reshape(1, N, 3)


# speedup vs baseline: 2.2837x; 1.2052x over previous
import functools

import jax
import jax.numpy as jnp
from jax import lax
from jax.experimental import pallas as pl
from jax.experimental.pallas import tpu as pltpu
from jax.experimental.pallas import tpu_sc as plsc

N = 50000
K = 6
D = 128
L = 5
IN_DIM = 3

BN = 1024
NP = 49 * BN
GRID = NP // BN

NC = 2
NS = 16
NW = NC * NS
NODES_PER_W = NP // NW
CHUNK = 16
NUM_CHUNKS = NODES_PER_W // CHUNK
ROWS_PER_CHUNK = CHUNK * K


def _gather_sum_sc(table, idx):
    mesh = plsc.VectorSubcoreMesh(core_axis_name="c", subcore_axis_name="s")

    @functools.partial(
        pl.kernel,
        mesh=mesh,
        out_type=jax.ShapeDtypeStruct((NP, D), jnp.float32),
        scratch_types=[
            pltpu.VMEM((ROWS_PER_CHUNK,), jnp.int32),
            pltpu.VMEM((ROWS_PER_CHUNK, D), jnp.float32),
            pltpu.VMEM((CHUNK, D), jnp.float32),
            pltpu.SemaphoreType.DMA,
        ],
    )
    def body(table_hbm, idx_hbm, out_hbm, idx_v, rows_v, acc_v, sem):
        wid = lax.axis_index("s") * NC + lax.axis_index("c")
        base = wid * NODES_PER_W

        def step(ci, carry):
            nb = base + ci * CHUNK
            pltpu.sync_copy(idx_hbm.at[pl.ds(nb * K, ROWS_PER_CHUNK)], idx_v)
            pltpu.async_copy(table_hbm.at[idx_v], rows_v, sem).wait()
            for nn in range(CHUNK):
                r0 = nn * K
                for c in range(0, D, 16):
                    acc = rows_v[r0, pl.ds(c, 16)]
                    for kk in range(1, K):
                        acc = acc + rows_v[r0 + kk, pl.ds(c, 16)]
                    acc_v[nn, pl.ds(c, 16)] = acc
            pltpu.sync_copy(acc_v, out_hbm.at[pl.ds(nb, CHUNK)])
            return carry

        lax.fori_loop(0, NUM_CHUNKS, step, 0)

    return body(table, idx)


def _ln(h, g, b):
    m = jnp.mean(h, axis=-1, keepdims=True)
    c = h - m
    v = jnp.mean(c * c, axis=-1, keepdims=True)
    return c * lax.rsqrt(v + 1e-5) * g + b


def _row_spec():
    return pl.BlockSpec((BN, D), lambda i: (i, 0))


def _full_spec(shape):
    nd = len(shape)
    return pl.BlockSpec(shape, lambda i: (0,) * nd)


def _emb_body(x_ref, We_ref, be_ref, g_ref, b_ref, h_ref, ln_ref):
    h = jnp.dot(x_ref[...], We_ref[...], preferred_element_type=jnp.float32)
    h = h + be_ref[...]
    h_ref[...] = h
    ln_ref[...] = _ln(h, g_ref[...], b_ref[...])


def _embed(xp, Wep, be, g0, b0):
    return pl.pallas_call(
        _emb_body,
        grid=(GRID,),
        in_specs=[
            pl.BlockSpec((BN, 8), lambda i: (i, 0)),
            _full_spec((8, D)),
            _full_spec((1, D)),
            _full_spec((1, D)),
            _full_spec((1, D)),
        ],
        out_specs=[_row_spec(), _row_spec()],
        out_shape=[
            jax.ShapeDtypeStruct((NP, D), jnp.float32),
            jax.ShapeDtypeStruct((NP, D), jnp.float32),
        ],
    )(xp, Wep, be, g0, b0)


def _mid_body(h_ref, s_ref, Wmp_ref, bmp_ref, g_ref, b_ref, Wqkv_ref,
              h2_ref, q_ref, kv_ref, ksum_ref):
    i = pl.program_id(0)
    s = s_ref[...] * (1.0 / K)
    h2 = h_ref[...] + jnp.dot(s, Wmp_ref[...],
                              preferred_element_type=jnp.float32)
    h2 = h2 + bmp_ref[...]
    h2_ref[...] = h2
    ln = _ln(h2, g_ref[...], b_ref[...])
    qkv = jnp.dot(ln, Wqkv_ref[...], preferred_element_type=jnp.float32)
    q = jax.nn.relu(qkv[:, :D]) + 1e-6
    k = jax.nn.relu(qkv[:, D:2 * D]) + 1e-6
    v = qkv[:, 2 * D:]
    rows = i * BN + lax.broadcasted_iota(jnp.int32, (BN, 1), 0)
    k = jnp.where(rows < N, k, 0.0)
    q_ref[...] = q.astype(jnp.bfloat16)
    kv_c = lax.dot_general(k, v, (((0,), (0,)), ((), ())),
                           preferred_element_type=jnp.float32)
    ksum_c = jnp.sum(k, axis=0, keepdims=True)

    @pl.when(i == 0)
    def _():
        kv_ref[...] = kv_c
        ksum_ref[...] = ksum_c

    @pl.when(i > 0)
    def _():
        kv_ref[...] += kv_c
        ksum_ref[...] += ksum_c


def _mid(h, s, Wmp_i, bmp_i, g, b, Wqkv_i):
    return pl.pallas_call(
        _mid_body,
        grid=(GRID,),
        in_specs=[
            _row_spec(), _row_spec(),
            _full_spec((D, D)), _full_spec((1, D)),
            _full_spec((1, D)), _full_spec((1, D)),
            _full_spec((D, 3 * D)),
        ],
        out_specs=[
            _row_spec(), _row_spec(),
            _full_spec((D, D)), _full_spec((1, D)),
        ],
        out_shape=[
            jax.ShapeDtypeStruct((NP, D), jnp.float32),
            jax.ShapeDtypeStruct((NP, D), jnp.bfloat16),
            jax.ShapeDtypeStruct((D, D), jnp.float32),
            jax.ShapeDtypeStruct((1, D), jnp.float32),
        ],
    )(h, s, Wmp_i, bmp_i, g, b, Wqkv_i)


def _attn_core(q, kv, ksum):
    q = q.astype(jnp.float32)
    z = 1.0 / (jnp.sum(q * ksum, axis=1, keepdims=True) + 1e-6)
    return jnp.dot(q, kv, preferred_element_type=jnp.float32) * z


def _post_body(h2_ref, q_ref, kv_ref, ksum_ref, Wout_ref, bout_ref, g_ref,
               b_ref, h3_ref, ln_ref):
    attn = _attn_core(q_ref[...], kv_ref[...], ksum_ref[...])
    h3 = h2_ref[...] + jnp.dot(attn, Wout_ref[...],
                               preferred_element_type=jnp.float32)
    h3 = h3 + bout_ref[...]
    h3_ref[...] = h3
    ln_ref[...] = _ln(h3, g_ref[...], b_ref[...])


def _post(h2, q, kv, ksum, Wout_i, bout_i, g_next, b_next):
    return pl.pallas_call(
        _post_body,
        grid=(GRID,),
        in_specs=[
            _row_spec(), _row_spec(),
            _full_spec((D, D)), _full_spec((1, D)),
            _full_spec((D, D)), _full_spec((1, D)),
            _full_spec((1, D)), _full_spec((1, D)),
        ],
        out_specs=[_row_spec(), _row_spec()],
        out_shape=[
            jax.ShapeDtypeStruct((NP, D), jnp.float32),
            jax.ShapeDtypeStruct((NP, D), jnp.float32),
        ],
    )(h2, q, kv, ksum, Wout_i, bout_i, g_next, b_next)


def _final_body(h2_ref, q_ref, kv_ref, ksum_ref, Wout_ref, bout_ref,
                Whead_ref, bhead_ref, out_ref):
    attn = _attn_core(q_ref[...], kv_ref[...], ksum_ref[...])
    h3 = h2_ref[...] + jnp.dot(attn, Wout_ref[...],
                               preferred_element_type=jnp.float32)
    h3 = h3 + bout_ref[...]
    out_ref[...] = jnp.dot(h3, Whead_ref[...],
                           preferred_element_type=jnp.float32) + bhead_ref[...]


def _final(h2, q, kv, ksum, Wout_i, bout_i, Whead_p, bhead_p):
    return pl.pallas_call(
        _final_body,
        grid=(GRID,),
        in_specs=[
            _row_spec(), _row_spec(),
            _full_spec((D, D)), _full_spec((1, D)),
            _full_spec((D, D)), _full_spec((1, D)),
            _full_spec((D, 8)), _full_spec((1, 8)),
        ],
        out_specs=[pl.BlockSpec((BN, 8), lambda i: (i, 0))],
        out_shape=[jax.ShapeDtypeStruct((NP, 8), jnp.float32)],
    )(h2, q, kv, ksum, Wout_i, bout_i, Whead_p, bhead_p)


def _gather_sum(table, idx):
    return _gather_sum_sc(table, idx)


def kernel(x, knn, W_emb, b_emb, ln_g, ln_b, Wmp, bmp, Wqkv, Wout, bout,
           Whead, bhead):
    x2 = x.reshape(N, IN_DIM)
    xp = jnp.pad(x2, ((0, NP - N), (0, 8 - IN_DIM)))
    Wep = jnp.pad(W_emb, ((0, 8 - IN_DIM), (0, 0)))
    idx = jnp.pad(knn.reshape(N * K), (0, (NP - N) * K))
    Whead_p = jnp.pad(Whead, ((0, 0), (0, 8 - Whead.shape[1])))
    bhead_p = jnp.pad(bhead, (0, 8 - bhead.shape[0])).reshape(1, 8)

    r = lambda a: a.reshape(1, D)
    h, ln1 = _embed(xp, Wep, b_emb.reshape(1, D), r(ln_g[0]), r(ln_b[0]))
    for i in range(L):
        s = _gather_sum(ln1, idx)
        h, q, kv, ksum = _mid(h, s, Wmp[i], r(bmp[i]),
                              r(ln_g[2 * i + 1]), r(ln_b[2 * i + 1]), Wqkv[i])
        if i + 1 < L:
            h, ln1 = _post(h, q, kv, ksum, Wout[i], r(bout[i]),
                           r(ln_g[2 * i + 2]), r(ln_b[2 * i + 2]))
        else:
            outp = _final(h, q, kv, ksum, Wout[i], r(bout[i]),
                          Whead_p, bhead_p)[0]
    return outp[:N, :3].reshape(1, N, 3)
